# baseline (speedup 1.0000x reference)
import jax
import jax.numpy as jnp
from jax.experimental import pallas as pl

B, N, NF, EF = 64, 64, 128, 4
HID, MSG, T, ENN_H, GATH, MLP_H, FADD = 128, 64, 3, 64, 128, 128, 32

_MPNN_KEYS = ('W_embed', 'enn_W1', 'enn_b1', 'enn_W2', 'enn_b2', 'W_msg',
              'gru_Wi', 'gru_Wh', 'gru_bi', 'gru_bh',
              'att_W1', 'att_b1', 'att_W2', 'att_b2',
              'emb_W1', 'emb_b1', 'emb_W2', 'emb_b2')
_PG_KEYS = _MPNN_KEYS + ('mlp1_W1', 'mlp1_b1', 'mlp1_W2', 'mlp1_b2',
                         'mlp2_W1', 'mlp2_b1', 'mlp2_W2', 'mlp2_b2')
_PC_KEYS = _MPNN_KEYS + ('out_W1', 'out_b1', 'out_W2', 'out_b2')


def _dot(a, b):
    return jnp.dot(a, b, preferred_element_type=jnp.float32)


def _gru_iters(h, gm3, p, streams):
    for _ in range(T):
        hj = _dot(h, p['W_msg'])
        if streams == 2:
            hj_pair = jnp.concatenate([hj[:N], hj[N:]], axis=1)
        else:
            hj_pair = hj
        m_pair = jnp.sum(gm3 * hj_pair[None, :, :], axis=1)
        if streams == 2:
            m = jnp.concatenate([m_pair[:, :MSG], m_pair[:, MSG:]], axis=0)
        else:
            m = m_pair
        gi = _dot(m, p['gru_Wi']) + p['gru_bi']
        gh = _dot(h, p['gru_Wh']) + p['gru_bh']
        z = jax.nn.sigmoid(gi[:, :HID] + gh[:, :HID])
        r = jax.nn.sigmoid(gi[:, HID:2 * HID] + gh[:, HID:2 * HID])
        nmsg = jnp.tanh(gi[:, 2 * HID:] + r * gh[:, 2 * HID:])
        h = (1.0 - z) * nmsg + z * h
    return h


def _mpnn_pair(Xp, nodes_stack, p, W1p, b1p, W2p, b2p):
    h = _dot(nodes_stack, p['W_embed'])
    emA = (jnp.sum(jnp.abs(Xp[:, :EF]), axis=-1, keepdims=True) > 1e-6)
    emB = (jnp.sum(jnp.abs(Xp[:, EF:]), axis=-1, keepdims=True) > 1e-6)
    lane = jax.lax.broadcasted_iota(jnp.int32, (1, 2 * MSG), 1)
    em = jnp.where(lane < MSG, emA.astype(jnp.float32), emB.astype(jnp.float32))
    a1 = jnp.maximum(_dot(Xp, W1p) + b1p, 0.0)
    gate = _dot(a1, W2p) + b2p
    gm3 = (gate * em).reshape(N, N, 2 * MSG)
    return _gru_iters(h, gm3, p, streams=2)


def _mpnn_single(X, nodes, p):
    h = _dot(nodes, p['W_embed'])
    em = (jnp.sum(jnp.abs(X), axis=-1, keepdims=True) > 1e-6).astype(jnp.float32)
    a1 = jnp.maximum(_dot(X, p['enn_W1']) + p['enn_b1'], 0.0)
    gate = _dot(a1, p['enn_W2']) + p['enn_b2']
    gm3 = (gate * em).reshape(N, N, MSG)
    return _gru_iters(h, gm3, p, streams=1)


def _tc_body(ln_ref, le_ref, fn_ref, xp_ref, w1p_ref, b1p_ref, w2p_ref, b2p_ref,
             *refs):
    npg, npc = len(_PG_KEYS), len(_PC_KEYS)
    pg = {k: refs[i][...] for i, k in enumerate(_PG_KEYS)}
    pc = {k: refs[npg + i][...] for i, k in enumerate(_PC_KEYS)}
    ea_ref, ec_ref, et_ref, idx_ref = refs[npg + npc:]

    ln = ln_ref[0]
    fn = fn_ref[0]
    Xl = le_ref[0]
    Xp = xp_ref[0]

    nodes_stack = jnp.concatenate([ln, fn], axis=0)
    h_stack = _mpnn_pair(Xp, nodes_stack, pg,
                         w1p_ref[...], b1p_ref[...], w2p_ref[...], b2p_ref[...])
    hl = h_stack[:N]

    cat = jnp.concatenate([h_stack, nodes_stack], axis=-1)
    att = jax.nn.sigmoid(
        _dot(jnp.maximum(_dot(cat, pg['att_W1']) + pg['att_b1'], 0.0),
             pg['att_W2']) + pg['att_b2'])
    emb = _dot(jnp.maximum(_dot(h_stack, pg['emb_W1']) + pg['emb_b1'], 0.0),
               pg['emb_W2']) + pg['emb_b2']
    ae = att * emb
    gl = jnp.sum(ae[:N], axis=0, keepdims=True)
    gf = jnp.sum(ae[N:], axis=0, keepdims=True)

    no = _dot(jnp.maximum(_dot(hl, pg['mlp1_W1']) + pg['mlp1_b1'], 0.0),
              pg['mlp1_W2']) + pg['mlp1_b2']
    na = no[:, :FADD]
    nc = no[:, FADD:FADD + EF]

    cat2 = jnp.concatenate([gl, gf], axis=-1)
    ft = _dot(jnp.maximum(_dot(cat2, pg['mlp2_W1']) + pg['mlp2_b1'], 0.0),
              pg['mlp2_W2']) + pg['mlp2_b2']

    mx = jnp.maximum(jnp.maximum(jnp.max(na), jnp.max(nc)), ft[0, 0])
    sa = jnp.exp(na - mx)
    sc = jnp.exp(nc - mx)
    st = jnp.exp(ft - mx)
    inv = 1.0 / (jnp.sum(sa) + jnp.sum(sc) + st[0, 0])
    ea_ref[0] = sa * inv
    ec_ref[0] = sc * inv
    et_ref[0] = st * inv

    hc = _mpnn_single(Xl, ln, pc)
    co = _dot(jnp.maximum(_dot(hc, pc['out_W1']) + pc['out_b1'], 0.0),
              pc['out_W2']) + pc['out_b2']
    iot = jax.lax.broadcasted_iota(jnp.int32, (N, 1), 0)
    m1 = jnp.max(co, axis=0, keepdims=True)
    i1 = jnp.min(jnp.where(co >= m1, iot, N), axis=0, keepdims=True)
    co2 = jnp.where(iot == i1, -jnp.inf, co)
    m2 = jnp.max(co2, axis=0, keepdims=True)
    i2 = jnp.min(jnp.where(co2 >= m2, iot, N), axis=0, keepdims=True)
    idx_ref[0] = jnp.concatenate([i1, i2], axis=1)


def kernel(linker_nodes, linker_edges, fragment_nodes, fragment_edges,
           params_gen, params_con):
    le = linker_edges.reshape(B, N * N, EF)
    fe = fragment_edges.reshape(B, N * N, EF)
    xp = jnp.concatenate([le, fe], axis=-1)

    z4 = jnp.zeros((EF, ENN_H), jnp.float32)
    zh = jnp.zeros((ENN_H, MSG), jnp.float32)
    w1 = params_gen['enn_W1']
    w2 = params_gen['enn_W2']
    w1p = jnp.concatenate([jnp.concatenate([w1, z4], 1),
                           jnp.concatenate([z4, w1], 1)], 0)
    w2p = jnp.concatenate([jnp.concatenate([w2, zh], 1),
                           jnp.concatenate([zh, w2], 1)], 0)
    b1p = jnp.concatenate([params_gen['enn_b1'],
                           params_gen['enn_b1']]).reshape(1, -1)
    b2p = jnp.concatenate([params_gen['enn_b2'],
                           params_gen['enn_b2']]).reshape(1, -1)

    def b2(x):
        return x.reshape(1, -1) if x.ndim == 1 else x

    wg = [b2(params_gen[k]) for k in _PG_KEYS]
    wc = [b2(params_con[k]) for k in _PC_KEYS]

    def bspec(shape):
        nd = len(shape)
        return pl.BlockSpec((1,) + shape[1:], lambda b: (b,) + (0,) * (nd - 1))

    def wspec(x):
        nd = x.ndim
        return pl.BlockSpec(x.shape, lambda b: (0,) * nd)

    in_specs = [bspec((B, N, NF)), bspec((B, N * N, EF)),
                bspec((B, N, NF)), bspec((B, N * N, 2 * EF))]
    in_specs += [wspec(x) for x in (w1p, b1p, w2p, b2p)]
    in_specs += [wspec(x) for x in wg + wc]

    out_shapes = [jax.ShapeDtypeStruct((B, N, FADD), jnp.float32),
                  jax.ShapeDtypeStruct((B, N, EF), jnp.float32),
                  jax.ShapeDtypeStruct((B, 1, 1), jnp.float32),
                  jax.ShapeDtypeStruct((B, 1, 2), jnp.int32)]
    out_specs = [bspec((B, N, FADD)), bspec((B, N, EF)),
                 bspec((B, 1, 1)), bspec((B, 1, 2))]

    ea, ec, et, idx = pl.pallas_call(
        _tc_body,
        grid=(B,),
        in_specs=in_specs,
        out_specs=out_specs,
        out_shape=out_shapes,
    )(linker_nodes, le, fragment_nodes, xp, w1p, b1p, w2p, b2p, *wg, *wc)

    apd = jnp.concatenate([ea.reshape(B, N * FADD), ec.reshape(B, N * EF),
                           et.reshape(B, 1)], axis=-1)
    two_idx = idx.reshape(B, 2)
    tanimoto = jnp.array(1.0, dtype=jnp.float32)
    return (apd, tanimoto, two_idx)


# speedup vs baseline: 1.0510x; 1.0510x over previous
import jax
import jax.numpy as jnp
from jax.experimental import pallas as pl

B, N, NF, EF = 64, 64, 128, 4
HID, MSG, T, ENN_H, GATH, MLP_H, FADD = 128, 64, 3, 64, 128, 128, 32

_MPNN_KEYS = ('W_embed', 'enn_W1', 'enn_b1', 'enn_W2', 'enn_b2', 'W_msg',
              'gru_Wi', 'gru_Wh', 'gru_bi', 'gru_bh',
              'att_W1', 'att_b1', 'att_W2', 'att_b2',
              'emb_W1', 'emb_b1', 'emb_W2', 'emb_b2')
_PG_KEYS = _MPNN_KEYS + ('mlp1_W1', 'mlp1_b1', 'mlp1_W2', 'mlp1_b2',
                         'mlp2_W1', 'mlp2_b1', 'mlp2_W2', 'mlp2_b2')
_PC_KEYS = _MPNN_KEYS + ('out_W1', 'out_b1', 'out_W2', 'out_b2')


def _dot(a, b):
    return jnp.dot(a, b, preferred_element_type=jnp.float32)


def _tree_sum_axis1(x):
    while x.shape[1] > 1:
        half = x.shape[1] // 2
        x = x[:, :half] + x[:, half:]
    return x[:, 0]


def _gru_iters(h, gm3, p, streams):
    for _ in range(T):
        hj = _dot(h, p['W_msg'])
        if streams == 2:
            hj_pair = jnp.concatenate([hj[:N], hj[N:]], axis=1)
        else:
            hj_pair = hj
        m_pair = _tree_sum_axis1(gm3 * hj_pair[None, :, :])
        if streams == 2:
            m = jnp.concatenate([m_pair[:, :MSG], m_pair[:, MSG:]], axis=0)
        else:
            m = m_pair
        gi = _dot(m, p['gru_Wi']) + p['gru_bi']
        gh = _dot(h, p['gru_Wh']) + p['gru_bh']
        z = jax.nn.sigmoid(gi[:, :HID] + gh[:, :HID])
        r = jax.nn.sigmoid(gi[:, HID:2 * HID] + gh[:, HID:2 * HID])
        nmsg = jnp.tanh(gi[:, 2 * HID:] + r * gh[:, 2 * HID:])
        h = (1.0 - z) * nmsg + z * h
    return h


def _mpnn_pair(Xp, nodes_stack, p, W1p, b1p, W2p, b2p, Msum):
    h = _dot(nodes_stack, p['W_embed'])
    em = (_dot(jnp.abs(Xp), Msum) > 1e-6).astype(jnp.float32)
    a1 = jnp.maximum(_dot(Xp, W1p) + b1p, 0.0)
    gate = _dot(a1, W2p) + b2p
    gm3 = (gate * em).reshape(N, N, 2 * MSG)
    return _gru_iters(h, gm3, p, streams=2)


def _mpnn_single(X, nodes, p, Msum1):
    h = _dot(nodes, p['W_embed'])
    em = (_dot(jnp.abs(X), Msum1) > 1e-6).astype(jnp.float32)
    a1 = jnp.maximum(_dot(X, p['enn_W1']) + p['enn_b1'], 0.0)
    gate = _dot(a1, p['enn_W2']) + p['enn_b2']
    gm3 = (gate * em).reshape(N, N, MSG)
    return _gru_iters(h, gm3, p, streams=1)


def _tc_body(ln_ref, le_ref, fn_ref, xp_ref, w1p_ref, b1p_ref, w2p_ref, b2p_ref,
             ms2_ref, ms1_ref, *refs):
    npg, npc = len(_PG_KEYS), len(_PC_KEYS)
    pg = {k: refs[i][...] for i, k in enumerate(_PG_KEYS)}
    pc = {k: refs[npg + i][...] for i, k in enumerate(_PC_KEYS)}
    ea_ref, ec_ref, et_ref, idx_ref = refs[npg + npc:]

    ln = ln_ref[0]
    fn = fn_ref[0]
    Xl = le_ref[0]
    Xp = xp_ref[0]

    nodes_stack = jnp.concatenate([ln, fn], axis=0)
    h_stack = _mpnn_pair(Xp, nodes_stack, pg,
                         w1p_ref[...], b1p_ref[...], w2p_ref[...], b2p_ref[...],
                         ms2_ref[...])
    hl = h_stack[:N]

    cat = jnp.concatenate([h_stack, nodes_stack], axis=-1)
    att = jax.nn.sigmoid(
        _dot(jnp.maximum(_dot(cat, pg['att_W1']) + pg['att_b1'], 0.0),
             pg['att_W2']) + pg['att_b2'])
    emb = _dot(jnp.maximum(_dot(h_stack, pg['emb_W1']) + pg['emb_b1'], 0.0),
               pg['emb_W2']) + pg['emb_b2']
    ae = att * emb
    gl = jnp.sum(ae[:N], axis=0, keepdims=True)
    gf = jnp.sum(ae[N:], axis=0, keepdims=True)

    no = _dot(jnp.maximum(_dot(hl, pg['mlp1_W1']) + pg['mlp1_b1'], 0.0),
              pg['mlp1_W2']) + pg['mlp1_b2']
    na = no[:, :FADD]
    nc = no[:, FADD:FADD + EF]

    cat2 = jnp.concatenate([gl, gf], axis=-1)
    ft = _dot(jnp.maximum(_dot(cat2, pg['mlp2_W1']) + pg['mlp2_b1'], 0.0),
              pg['mlp2_W2']) + pg['mlp2_b2']

    mx = jnp.maximum(jnp.maximum(jnp.max(na), jnp.max(nc)), ft[0, 0])
    sa = jnp.exp(na - mx)
    sc = jnp.exp(nc - mx)
    st = jnp.exp(ft - mx)
    inv = 1.0 / (jnp.sum(sa) + jnp.sum(sc) + st[0, 0])
    ea_ref[0] = sa * inv
    ec_ref[0] = sc * inv
    et_ref[0] = st * inv

    hc = _mpnn_single(Xl, ln, pc, ms1_ref[...])
    co = _dot(jnp.maximum(_dot(hc, pc['out_W1']) + pc['out_b1'], 0.0),
              pc['out_W2']) + pc['out_b2']
    iot = jax.lax.broadcasted_iota(jnp.int32, (N, 1), 0)
    m1 = jnp.max(co, axis=0, keepdims=True)
    i1 = jnp.min(jnp.where(co >= m1, iot, N), axis=0, keepdims=True)
    co2 = jnp.where(iot == i1, -jnp.inf, co)
    m2 = jnp.max(co2, axis=0, keepdims=True)
    i2 = jnp.min(jnp.where(co2 >= m2, iot, N), axis=0, keepdims=True)
    idx_ref[0] = jnp.concatenate([i1, i2], axis=1)


def kernel(linker_nodes, linker_edges, fragment_nodes, fragment_edges,
           params_gen, params_con):
    le = linker_edges.reshape(B, N * N, EF)
    fe = fragment_edges.reshape(B, N * N, EF)
    xp = jnp.concatenate([le, fe], axis=-1)

    z4 = jnp.zeros((EF, ENN_H), jnp.float32)
    zh = jnp.zeros((ENN_H, MSG), jnp.float32)
    w1 = params_gen['enn_W1']
    w2 = params_gen['enn_W2']
    w1p = jnp.concatenate([jnp.concatenate([w1, z4], 1),
                           jnp.concatenate([z4, w1], 1)], 0)
    w2p = jnp.concatenate([jnp.concatenate([w2, zh], 1),
                           jnp.concatenate([zh, w2], 1)], 0)
    b1p = jnp.concatenate([params_gen['enn_b1'],
                           params_gen['enn_b1']]).reshape(1, -1)
    b2p = jnp.concatenate([params_gen['enn_b2'],
                           params_gen['enn_b2']]).reshape(1, -1)

    o44 = jnp.ones((EF, MSG), jnp.float32)
    z44 = jnp.zeros((EF, MSG), jnp.float32)
    ms2 = jnp.concatenate([jnp.concatenate([o44, z44], 1),
                           jnp.concatenate([z44, o44], 1)], 0)
    ms1 = o44

    def b2(x):
        return x.reshape(1, -1) if x.ndim == 1 else x

    wg = [b2(params_gen[k]) for k in _PG_KEYS]
    wc = [b2(params_con[k]) for k in _PC_KEYS]

    def bspec(shape):
        nd = len(shape)
        return pl.BlockSpec((1,) + shape[1:], lambda b: (b,) + (0,) * (nd - 1))

    def wspec(x):
        nd = x.ndim
        return pl.BlockSpec(x.shape, lambda b: (0,) * nd)

    in_specs = [bspec((B, N, NF)), bspec((B, N * N, EF)),
                bspec((B, N, NF)), bspec((B, N * N, 2 * EF))]
    in_specs += [wspec(x) for x in (w1p, b1p, w2p, b2p, ms2, ms1)]
    in_specs += [wspec(x) for x in wg + wc]

    out_shapes = [jax.ShapeDtypeStruct((B, N, FADD), jnp.float32),
                  jax.ShapeDtypeStruct((B, N, EF), jnp.float32),
                  jax.ShapeDtypeStruct((B, 1, 1), jnp.float32),
                  jax.ShapeDtypeStruct((B, 1, 2), jnp.int32)]
    out_specs = [bspec((B, N, FADD)), bspec((B, N, EF)),
                 bspec((B, 1, 1)), bspec((B, 1, 2))]

    ea, ec, et, idx = pl.pallas_call(
        _tc_body,
        grid=(B,),
        in_specs=in_specs,
        out_specs=out_specs,
        out_shape=out_shapes,
    )(linker_nodes, le, fragment_nodes, xp, w1p, b1p, w2p, b2p, ms2, ms1,
      *wg, *wc)

    apd = jnp.concatenate([ea.reshape(B, N * FADD), ec.reshape(B, N * EF),
                           et.reshape(B, 1)], axis=-1)
    two_idx = idx.reshape(B, 2)
    tanimoto = jnp.array(1.0, dtype=jnp.float32)
    return (apd, tanimoto, two_idx)


# speedup vs baseline: 1.2402x; 1.1800x over previous
import jax
import jax.numpy as jnp
from jax.experimental import pallas as pl

B, N, NF, EF = 64, 64, 128, 4
HID, MSG, T, ENN_H, GATH, MLP_H, FADD = 128, 64, 3, 64, 128, 128, 32

_MPNN_KEYS = ('W_embed', 'enn_W1', 'enn_b1', 'enn_W2', 'enn_b2', 'W_msg',
              'gru_Wi', 'gru_Wh', 'gru_bi', 'gru_bh',
              'att_W1', 'att_b1', 'att_W2', 'att_b2',
              'emb_W1', 'emb_b1', 'emb_W2', 'emb_b2')
_PG_KEYS = _MPNN_KEYS + ('mlp1_W1', 'mlp1_b1', 'mlp1_W2', 'mlp1_b2',
                         'mlp2_W1', 'mlp2_b1', 'mlp2_W2', 'mlp2_b2')
_PC_KEYS = _MPNN_KEYS + ('out_W1', 'out_b1', 'out_W2', 'out_b2')


def _dot(a, b):
    return jnp.dot(a, b, preferred_element_type=jnp.float32)


def _tree_sum_axis1(x):
    while x.shape[1] > 1:
        half = x.shape[1] // 2
        x = x[:, :half] + x[:, half:]
    return x[:, 0]


def _gru_iters(h, gm3, p, streams):
    for _ in range(T):
        hj = _dot(h, p['W_msg'])
        if streams == 2:
            hj_pair = jnp.concatenate([hj[:N], hj[N:]], axis=1)
        else:
            hj_pair = hj
        m_pair = _tree_sum_axis1(gm3 * hj_pair[None, :, :])
        if streams == 2:
            m = jnp.concatenate([m_pair[:, :MSG], m_pair[:, MSG:]], axis=0)
        else:
            m = m_pair
        gi = _dot(m, p['gru_Wi']) + p['gru_bi']
        gh = _dot(h, p['gru_Wh']) + p['gru_bh']
        z = jax.nn.sigmoid(gi[:, :HID] + gh[:, :HID])
        r = jax.nn.sigmoid(gi[:, HID:2 * HID] + gh[:, HID:2 * HID])
        nmsg = jnp.tanh(gi[:, 2 * HID:] + r * gh[:, 2 * HID:])
        h = (1.0 - z) * nmsg + z * h
    return h


def _mpnn_pair(Xp, nodes_stack, p, W1p, b1p, W2p, b2p, Msum):
    h = _dot(nodes_stack, p['W_embed'])
    em = (_dot(jnp.abs(Xp), Msum) > 1e-6).astype(jnp.float32)
    a1 = jnp.maximum(_dot(Xp, W1p) + b1p, 0.0)
    gate = _dot(a1, W2p) + b2p
    gm3 = (gate * em).reshape(N, N, 2 * MSG)
    return _gru_iters(h, gm3, p, streams=2)


def _mpnn_single(X, nodes, p, Msum1):
    h = _dot(nodes, p['W_embed'])
    em = (_dot(jnp.abs(X), Msum1) > 1e-6).astype(jnp.float32)
    a1 = jnp.maximum(_dot(X, p['enn_W1']) + p['enn_b1'], 0.0)
    gate = _dot(a1, p['enn_W2']) + p['enn_b2']
    gm3 = (gate * em).reshape(N, N, MSG)
    return _gru_iters(h, gm3, p, streams=1)


def _tc_body(ln_ref, le_ref, fn_ref, xp_ref, w1p_ref, b1p_ref, w2p_ref, b2p_ref,
             ms2_ref, ms1_ref, *refs):
    npg, npc = len(_PG_KEYS), len(_PC_KEYS)
    pg = {k: refs[i][...] for i, k in enumerate(_PG_KEYS)}
    pc = {k: refs[npg + i][...] for i, k in enumerate(_PC_KEYS)}
    ea_ref, ec_ref, et_ref, idx_ref = refs[npg + npc:]

    ln = ln_ref[0]
    fn = fn_ref[0]
    Xl = le_ref[0].reshape(N * N, EF)
    Xf = xp_ref[0].reshape(N * N, EF)
    Xp = jnp.concatenate([Xl, Xf], axis=1)

    nodes_stack = jnp.concatenate([ln, fn], axis=0)
    h_stack = _mpnn_pair(Xp, nodes_stack, pg,
                         w1p_ref[...], b1p_ref[...], w2p_ref[...], b2p_ref[...],
                         ms2_ref[...])
    hl = h_stack[:N]

    cat = jnp.concatenate([h_stack, nodes_stack], axis=-1)
    att = jax.nn.sigmoid(
        _dot(jnp.maximum(_dot(cat, pg['att_W1']) + pg['att_b1'], 0.0),
             pg['att_W2']) + pg['att_b2'])
    emb = _dot(jnp.maximum(_dot(h_stack, pg['emb_W1']) + pg['emb_b1'], 0.0),
               pg['emb_W2']) + pg['emb_b2']
    ae = att * emb
    gl = jnp.sum(ae[:N], axis=0, keepdims=True)
    gf = jnp.sum(ae[N:], axis=0, keepdims=True)

    no = _dot(jnp.maximum(_dot(hl, pg['mlp1_W1']) + pg['mlp1_b1'], 0.0),
              pg['mlp1_W2']) + pg['mlp1_b2']
    na = no[:, :FADD]
    nc = no[:, FADD:FADD + EF]

    cat2 = jnp.concatenate([gl, gf], axis=-1)
    ft = _dot(jnp.maximum(_dot(cat2, pg['mlp2_W1']) + pg['mlp2_b1'], 0.0),
              pg['mlp2_W2']) + pg['mlp2_b2']

    mx = jnp.maximum(jnp.maximum(jnp.max(na), jnp.max(nc)), ft[0, 0])
    sa = jnp.exp(na - mx)
    sc = jnp.exp(nc - mx)
    st = jnp.exp(ft - mx)
    inv = 1.0 / (jnp.sum(sa) + jnp.sum(sc) + st[0, 0])
    ea_ref[0] = sa * inv
    ec_ref[0] = sc * inv
    et_ref[0] = st * inv

    hc = _mpnn_single(Xl, ln, pc, ms1_ref[...])
    co = _dot(jnp.maximum(_dot(hc, pc['out_W1']) + pc['out_b1'], 0.0),
              pc['out_W2']) + pc['out_b2']
    iot = jax.lax.broadcasted_iota(jnp.int32, (N, 1), 0)
    m1 = jnp.max(co, axis=0, keepdims=True)
    i1 = jnp.min(jnp.where(co >= m1, iot, N), axis=0, keepdims=True)
    co2 = jnp.where(iot == i1, -jnp.inf, co)
    m2 = jnp.max(co2, axis=0, keepdims=True)
    i2 = jnp.min(jnp.where(co2 >= m2, iot, N), axis=0, keepdims=True)
    idx_ref[0] = jnp.concatenate([i1, i2], axis=1)


def kernel(linker_nodes, linker_edges, fragment_nodes, fragment_edges,
           params_gen, params_con):
    z4 = jnp.zeros((EF, ENN_H), jnp.float32)
    zh = jnp.zeros((ENN_H, MSG), jnp.float32)
    w1 = params_gen['enn_W1']
    w2 = params_gen['enn_W2']
    w1p = jnp.concatenate([jnp.concatenate([w1, z4], 1),
                           jnp.concatenate([z4, w1], 1)], 0)
    w2p = jnp.concatenate([jnp.concatenate([w2, zh], 1),
                           jnp.concatenate([zh, w2], 1)], 0)
    b1p = jnp.concatenate([params_gen['enn_b1'],
                           params_gen['enn_b1']]).reshape(1, -1)
    b2p = jnp.concatenate([params_gen['enn_b2'],
                           params_gen['enn_b2']]).reshape(1, -1)

    o44 = jnp.ones((EF, MSG), jnp.float32)
    z44 = jnp.zeros((EF, MSG), jnp.float32)
    ms2 = jnp.concatenate([jnp.concatenate([o44, z44], 1),
                           jnp.concatenate([z44, o44], 1)], 0)
    ms1 = o44

    def b2(x):
        return x.reshape(1, -1) if x.ndim == 1 else x

    wg = [b2(params_gen[k]) for k in _PG_KEYS]
    wc = [b2(params_con[k]) for k in _PC_KEYS]

    def bspec(shape):
        nd = len(shape)
        return pl.BlockSpec((1,) + shape[1:], lambda b: (b,) + (0,) * (nd - 1))

    def wspec(x):
        nd = x.ndim
        return pl.BlockSpec(x.shape, lambda b: (0,) * nd)

    in_specs = [bspec((B, N, NF)), bspec((B, N, N, EF)),
                bspec((B, N, NF)), bspec((B, N, N, EF))]
    in_specs += [wspec(x) for x in (w1p, b1p, w2p, b2p, ms2, ms1)]
    in_specs += [wspec(x) for x in wg + wc]

    out_shapes = [jax.ShapeDtypeStruct((B, N, FADD), jnp.float32),
                  jax.ShapeDtypeStruct((B, N, EF), jnp.float32),
                  jax.ShapeDtypeStruct((B, 1, 1), jnp.float32),
                  jax.ShapeDtypeStruct((B, 1, 2), jnp.int32)]
    out_specs = [bspec((B, N, FADD)), bspec((B, N, EF)),
                 bspec((B, 1, 1)), bspec((B, 1, 2))]

    ea, ec, et, idx = pl.pallas_call(
        _tc_body,
        grid=(B,),
        in_specs=in_specs,
        out_specs=out_specs,
        out_shape=out_shapes,
    )(linker_nodes, linker_edges, fragment_nodes, fragment_edges,
      w1p, b1p, w2p, b2p, ms2, ms1, *wg, *wc)

    apd = jnp.concatenate([ea.reshape(B, N * FADD), ec.reshape(B, N * EF),
                           et.reshape(B, 1)], axis=-1)
    two_idx = idx.reshape(B, 2)
    tanimoto = jnp.array(1.0, dtype=jnp.float32)
    return (apd, tanimoto, two_idx)


# speedup vs baseline: 1.2450x; 1.0039x over previous
import jax
import jax.numpy as jnp
from jax.experimental import pallas as pl

B, N, NF, EF = 64, 64, 128, 4
HID, MSG, T, ENN_H, GATH, MLP_H, FADD = 128, 64, 3, 64, 128, 128, 32

_MPNN_KEYS = ('W_embed', 'enn_W1', 'enn_b1', 'enn_W2', 'enn_b2', 'W_msg',
              'gru_Wi', 'gru_Wh', 'gru_bi', 'gru_bh',
              'att_W1', 'att_b1', 'att_W2', 'att_b2',
              'emb_W1', 'emb_b1', 'emb_W2', 'emb_b2')
_PG_KEYS = _MPNN_KEYS + ('mlp1_W1', 'mlp1_b1', 'mlp1_W2', 'mlp1_b2',
                         'mlp2_W1', 'mlp2_b1', 'mlp2_W2', 'mlp2_b2')
_PC_KEYS = _MPNN_KEYS + ('out_W1', 'out_b1', 'out_W2', 'out_b2')


def _dot(a, b):
    return jnp.dot(a, b, preferred_element_type=jnp.float32)


def _tree_sum_axis1(x):
    while x.shape[1] > 1:
        half = x.shape[1] // 2
        x = x[:, :half] + x[:, half:]
    return x[:, 0]


def _gru_iters(h, gm3, p, streams):
    for _ in range(T):
        hj = _dot(h, p['W_msg'])
        if streams == 2:
            hj_pair = jnp.concatenate([hj[:N], hj[N:]], axis=1)
        else:
            hj_pair = hj
        m_pair = _tree_sum_axis1(gm3 * hj_pair[None, :, :])
        if streams == 2:
            m = jnp.concatenate([m_pair[:, :MSG], m_pair[:, MSG:]], axis=0)
        else:
            m = m_pair
        gi = _dot(m, p['gru_Wi']) + p['gru_bi']
        gh = _dot(h, p['gru_Wh']) + p['gru_bh']
        z = jax.nn.sigmoid(gi[:, :HID] + gh[:, :HID])
        r = jax.nn.sigmoid(gi[:, HID:2 * HID] + gh[:, HID:2 * HID])
        nmsg = jnp.tanh(gi[:, 2 * HID:] + r * gh[:, 2 * HID:])
        h = (1.0 - z) * nmsg + z * h
    return h


def _mpnn_pair(Xp, nodes_stack, p, W1p, b1p, W2p, b2p, Msum):
    h = _dot(nodes_stack, p['W_embed'])
    em = (_dot(jnp.abs(Xp), Msum) > 1e-6).astype(jnp.float32)
    a1 = jnp.maximum(_dot(Xp, W1p) + b1p, 0.0)
    gate = _dot(a1, W2p) + b2p
    gm3 = (gate * em).reshape(N, N, 2 * MSG)
    return _gru_iters(h, gm3, p, streams=2)


def _mpnn_single(X, nodes, p, Msum1):
    h = _dot(nodes, p['W_embed'])
    em = (_dot(jnp.abs(X), Msum1) > 1e-6).astype(jnp.float32)
    a1 = jnp.maximum(_dot(X, p['enn_W1']) + p['enn_b1'], 0.0)
    gate = _dot(a1, p['enn_W2']) + p['enn_b2']
    gm3 = (gate * em).reshape(N, N, MSG)
    return _gru_iters(h, gm3, p, streams=1)


def _tc_body(ln_ref, le_ref, fn_ref, xp_ref, *refs):
    npg, npc = len(_PG_KEYS), len(_PC_KEYS)
    pg = {k: refs[i][...] for i, k in enumerate(_PG_KEYS)}
    pc = {k: refs[npg + i][...] for i, k in enumerate(_PC_KEYS)}
    ea_ref, ec_ref, et_ref, idx_ref = refs[npg + npc:]

    ln = ln_ref[0]
    fn = fn_ref[0]
    Xl = le_ref[0].reshape(N * N, EF)
    Xf = xp_ref[0].reshape(N * N, EF)
    Xp = jnp.concatenate([Xl, Xf], axis=1)

    z4 = jnp.zeros((EF, ENN_H), jnp.float32)
    zh = jnp.zeros((ENN_H, MSG), jnp.float32)
    w1p = jnp.concatenate([jnp.concatenate([pg['enn_W1'], z4], 1),
                           jnp.concatenate([z4, pg['enn_W1']], 1)], 0)
    w2p = jnp.concatenate([jnp.concatenate([pg['enn_W2'], zh], 1),
                           jnp.concatenate([zh, pg['enn_W2']], 1)], 0)
    b1p = jnp.concatenate([pg['enn_b1'], pg['enn_b1']], axis=1)
    b2p = jnp.concatenate([pg['enn_b2'], pg['enn_b2']], axis=1)
    o44 = jnp.ones((EF, MSG), jnp.float32)
    z44 = jnp.zeros((EF, MSG), jnp.float32)
    ms2 = jnp.concatenate([jnp.concatenate([o44, z44], 1),
                           jnp.concatenate([z44, o44], 1)], 0)

    nodes_stack = jnp.concatenate([ln, fn], axis=0)
    h_stack = _mpnn_pair(Xp, nodes_stack, pg, w1p, b1p, w2p, b2p, ms2)
    hl = h_stack[:N]

    cat = jnp.concatenate([h_stack, nodes_stack], axis=-1)
    att = jax.nn.sigmoid(
        _dot(jnp.maximum(_dot(cat, pg['att_W1']) + pg['att_b1'], 0.0),
             pg['att_W2']) + pg['att_b2'])
    emb = _dot(jnp.maximum(_dot(h_stack, pg['emb_W1']) + pg['emb_b1'], 0.0),
               pg['emb_W2']) + pg['emb_b2']
    ae = att * emb
    gl = jnp.sum(ae[:N], axis=0, keepdims=True)
    gf = jnp.sum(ae[N:], axis=0, keepdims=True)

    no = _dot(jnp.maximum(_dot(hl, pg['mlp1_W1']) + pg['mlp1_b1'], 0.0),
              pg['mlp1_W2']) + pg['mlp1_b2']
    na = no[:, :FADD]
    nc = no[:, FADD:FADD + EF]

    cat2 = jnp.concatenate([gl, gf], axis=-1)
    ft = _dot(jnp.maximum(_dot(cat2, pg['mlp2_W1']) + pg['mlp2_b1'], 0.0),
              pg['mlp2_W2']) + pg['mlp2_b2']

    mx = jnp.maximum(jnp.maximum(jnp.max(na), jnp.max(nc)), ft[0, 0])
    sa = jnp.exp(na - mx)
    sc = jnp.exp(nc - mx)
    st = jnp.exp(ft - mx)
    inv = 1.0 / (jnp.sum(sa) + jnp.sum(sc) + st[0, 0])
    ea_ref[0] = sa * inv
    ec_ref[0] = sc * inv
    et_ref[0] = st * inv

    hc = _mpnn_single(Xl, ln, pc, o44)
    co = _dot(jnp.maximum(_dot(hc, pc['out_W1']) + pc['out_b1'], 0.0),
              pc['out_W2']) + pc['out_b2']
    iot = jax.lax.broadcasted_iota(jnp.int32, (N, 1), 0)
    m1 = jnp.max(co, axis=0, keepdims=True)
    i1 = jnp.min(jnp.where(co >= m1, iot, N), axis=0, keepdims=True)
    co2 = jnp.where(iot == i1, -jnp.inf, co)
    m2 = jnp.max(co2, axis=0, keepdims=True)
    i2 = jnp.min(jnp.where(co2 >= m2, iot, N), axis=0, keepdims=True)
    idx_ref[0] = jnp.concatenate([i1, i2], axis=1)


def kernel(linker_nodes, linker_edges, fragment_nodes, fragment_edges,
           params_gen, params_con):
    def b2(x):
        return x.reshape(1, -1) if x.ndim == 1 else x

    wg = [b2(params_gen[k]) for k in _PG_KEYS]
    wc = [b2(params_con[k]) for k in _PC_KEYS]

    def bspec(shape):
        nd = len(shape)
        return pl.BlockSpec((1,) + shape[1:], lambda b: (b,) + (0,) * (nd - 1))

    def wspec(x):
        nd = x.ndim
        return pl.BlockSpec(x.shape, lambda b: (0,) * nd)

    in_specs = [bspec((B, N, NF)), bspec((B, N, N, EF)),
                bspec((B, N, NF)), bspec((B, N, N, EF))]
    in_specs += [wspec(x) for x in wg + wc]

    out_shapes = [jax.ShapeDtypeStruct((B, N, FADD), jnp.float32),
                  jax.ShapeDtypeStruct((B, N, EF), jnp.float32),
                  jax.ShapeDtypeStruct((B, 1, 1), jnp.float32),
                  jax.ShapeDtypeStruct((B, 1, 2), jnp.int32)]
    out_specs = [bspec((B, N, FADD)), bspec((B, N, EF)),
                 bspec((B, 1, 1)), bspec((B, 1, 2))]

    ea, ec, et, idx = pl.pallas_call(
        _tc_body,
        grid=(B,),
        in_specs=in_specs,
        out_specs=out_specs,
        out_shape=out_shapes,
    )(linker_nodes, linker_edges, fragment_nodes, fragment_edges, *wg, *wc)

    apd = jnp.concatenate([ea.reshape(B, N * FADD), ec.reshape(B, N * EF),
                           et.reshape(B, 1)], axis=-1)
    two_idx = idx.reshape(B, 2)
    tanimoto = jnp.array(1.0, dtype=jnp.float32)
    return (apd, tanimoto, two_idx)


# speedup vs baseline: 1.2454x; 1.0003x over previous
import jax
import jax.numpy as jnp
from jax.experimental import pallas as pl

B, N, NF, EF = 64, 64, 128, 4
HID, MSG, T, ENN_H, GATH, MLP_H, FADD = 128, 64, 3, 64, 128, 128, 32

_MPNN_KEYS = ('W_embed', 'enn_W1', 'enn_b1', 'enn_W2', 'enn_b2', 'W_msg',
              'gru_Wi', 'gru_Wh', 'gru_bi', 'gru_bh',
              'att_W1', 'att_b1', 'att_W2', 'att_b2',
              'emb_W1', 'emb_b1', 'emb_W2', 'emb_b2')
_PG_KEYS = _MPNN_KEYS + ('mlp1_W1', 'mlp1_b1', 'mlp1_W2', 'mlp1_b2',
                         'mlp2_W1', 'mlp2_b1', 'mlp2_W2', 'mlp2_b2')
_PC_KEYS = _MPNN_KEYS + ('out_W1', 'out_b1', 'out_W2', 'out_b2')


def _dot(a, b):
    return jnp.dot(a, b, preferred_element_type=jnp.float32)


def _tree_sum_axis1(x):
    while x.shape[1] > 1:
        half = x.shape[1] // 2
        x = x[:, :half] + x[:, half:]
    return x[:, 0]


def _gru_iters(h, gm3, p, streams):
    for _ in range(T):
        hj = _dot(h, p['W_msg'])
        if streams == 2:
            hj_pair = jnp.concatenate([hj[:N], hj[N:]], axis=1)
        else:
            hj_pair = hj
        m_pair = _tree_sum_axis1(gm3 * hj_pair[None, :, :])
        if streams == 2:
            m = jnp.concatenate([m_pair[:, :MSG], m_pair[:, MSG:]], axis=0)
        else:
            m = m_pair
        gi = _dot(m, p['gru_Wi']) + p['gru_bi']
        gh = _dot(h, p['gru_Wh']) + p['gru_bh']
        z = jax.nn.sigmoid(gi[:, :HID] + gh[:, :HID])
        r = jax.nn.sigmoid(gi[:, HID:2 * HID] + gh[:, HID:2 * HID])
        nmsg = jnp.tanh(gi[:, 2 * HID:] + r * gh[:, 2 * HID:])
        h = (1.0 - z) * nmsg + z * h
    return h


def _mpnn_pair(Xp, nodes_stack, p, W1p, b1p, W2p, b2p, Msum):
    h = _dot(nodes_stack, p['W_embed'])
    em = (_dot(jnp.abs(Xp), Msum) > 1e-6).astype(jnp.float32)
    a1 = jnp.maximum(_dot(Xp, W1p) + b1p, 0.0)
    gate = _dot(a1, W2p) + b2p
    gm3 = (gate * em).reshape(N, N, 2 * MSG)
    return _gru_iters(h, gm3, p, streams=2)


def _mpnn_single(X, nodes, p, Msum1):
    h = _dot(nodes, p['W_embed'])
    em = (_dot(jnp.abs(X), Msum1) > 1e-6).astype(jnp.float32)
    a1 = jnp.maximum(_dot(X, p['enn_W1']) + p['enn_b1'], 0.0)
    gate = _dot(a1, p['enn_W2']) + p['enn_b2']
    gm3 = (gate * em).reshape(N, N, MSG)
    return _gru_iters(h, gm3, p, streams=1)


def _tc_body(ln_ref, le_ref, fn_ref, xp_ref, *refs):
    npg, npc = len(_PG_KEYS), len(_PC_KEYS)
    pg = {k: refs[i][...] for i, k in enumerate(_PG_KEYS)}
    pc = {k: refs[npg + i][...] for i, k in enumerate(_PC_KEYS)}
    apd_ref, idx_ref = refs[npg + npc:]

    ln = ln_ref[0]
    fn = fn_ref[0]
    Xl = le_ref[0].reshape(N * N, EF)
    Xf = xp_ref[0].reshape(N * N, EF)
    Xp = jnp.concatenate([Xl, Xf], axis=1)

    z4 = jnp.zeros((EF, ENN_H), jnp.float32)
    zh = jnp.zeros((ENN_H, MSG), jnp.float32)
    w1p = jnp.concatenate([jnp.concatenate([pg['enn_W1'], z4], 1),
                           jnp.concatenate([z4, pg['enn_W1']], 1)], 0)
    w2p = jnp.concatenate([jnp.concatenate([pg['enn_W2'], zh], 1),
                           jnp.concatenate([zh, pg['enn_W2']], 1)], 0)
    b1p = jnp.concatenate([pg['enn_b1'], pg['enn_b1']], axis=1)
    b2p = jnp.concatenate([pg['enn_b2'], pg['enn_b2']], axis=1)
    o44 = jnp.ones((EF, MSG), jnp.float32)
    z44 = jnp.zeros((EF, MSG), jnp.float32)
    ms2 = jnp.concatenate([jnp.concatenate([o44, z44], 1),
                           jnp.concatenate([z44, o44], 1)], 0)

    nodes_stack = jnp.concatenate([ln, fn], axis=0)
    h_stack = _mpnn_pair(Xp, nodes_stack, pg, w1p, b1p, w2p, b2p, ms2)
    hl = h_stack[:N]

    cat = jnp.concatenate([h_stack, nodes_stack], axis=-1)
    att = jax.nn.sigmoid(
        _dot(jnp.maximum(_dot(cat, pg['att_W1']) + pg['att_b1'], 0.0),
             pg['att_W2']) + pg['att_b2'])
    emb = _dot(jnp.maximum(_dot(h_stack, pg['emb_W1']) + pg['emb_b1'], 0.0),
               pg['emb_W2']) + pg['emb_b2']
    ae = att * emb
    gl = jnp.sum(ae[:N], axis=0, keepdims=True)
    gf = jnp.sum(ae[N:], axis=0, keepdims=True)

    no = _dot(jnp.maximum(_dot(hl, pg['mlp1_W1']) + pg['mlp1_b1'], 0.0),
              pg['mlp1_W2']) + pg['mlp1_b2']
    na = no[:, :FADD]
    nc = no[:, FADD:FADD + EF]

    cat2 = jnp.concatenate([gl, gf], axis=-1)
    ft = _dot(jnp.maximum(_dot(cat2, pg['mlp2_W1']) + pg['mlp2_b1'], 0.0),
              pg['mlp2_W2']) + pg['mlp2_b2']

    mx = jnp.maximum(jnp.maximum(jnp.max(na), jnp.max(nc)), ft[0, 0])
    sa = jnp.exp(na - mx)
    sc = jnp.exp(nc - mx)
    st = jnp.exp(ft - mx)
    inv = 1.0 / (jnp.sum(sa) + jnp.sum(sc) + st[0, 0])
    ea = sa * inv
    ec = sc * inv
    for i in range(N):
        apd_ref[0, :, FADD * i:FADD * (i + 1)] = ea[i:i + 1, :]
        apd_ref[0, :, N * FADD + EF * i:N * FADD + EF * (i + 1)] = ec[i:i + 1, :]
    apd_ref[0, :, N * FADD + N * EF:] = st * inv

    hc = _mpnn_single(Xl, ln, pc, o44)
    co = _dot(jnp.maximum(_dot(hc, pc['out_W1']) + pc['out_b1'], 0.0),
              pc['out_W2']) + pc['out_b2']
    iot = jax.lax.broadcasted_iota(jnp.int32, (N, 1), 0)
    m1 = jnp.max(co, axis=0, keepdims=True)
    i1 = jnp.min(jnp.where(co >= m1, iot, N), axis=0, keepdims=True)
    co2 = jnp.where(iot == i1, -jnp.inf, co)
    m2 = jnp.max(co2, axis=0, keepdims=True)
    i2 = jnp.min(jnp.where(co2 >= m2, iot, N), axis=0, keepdims=True)
    idx_ref[0] = jnp.concatenate([i1, i2], axis=1)


def kernel(linker_nodes, linker_edges, fragment_nodes, fragment_edges,
           params_gen, params_con):
    def b2(x):
        return x.reshape(1, -1) if x.ndim == 1 else x

    wg = [b2(params_gen[k]) for k in _PG_KEYS]
    wc = [b2(params_con[k]) for k in _PC_KEYS]

    def bspec(shape):
        nd = len(shape)
        return pl.BlockSpec((1,) + shape[1:], lambda b: (b,) + (0,) * (nd - 1))

    def wspec(x):
        nd = x.ndim
        return pl.BlockSpec(x.shape, lambda b: (0,) * nd)

    in_specs = [bspec((B, N, NF)), bspec((B, N, N, EF)),
                bspec((B, N, NF)), bspec((B, N, N, EF))]
    in_specs += [wspec(x) for x in wg + wc]

    nout = N * FADD + N * EF + 1
    out_shapes = [jax.ShapeDtypeStruct((B, 1, nout), jnp.float32),
                  jax.ShapeDtypeStruct((B, 1, 2), jnp.int32)]
    out_specs = [bspec((B, 1, nout)), bspec((B, 1, 2))]

    apd, idx = pl.pallas_call(
        _tc_body,
        grid=(B,),
        in_specs=in_specs,
        out_specs=out_specs,
        out_shape=out_shapes,
    )(linker_nodes, linker_edges, fragment_nodes, fragment_edges, *wg, *wc)

    two_idx = idx.reshape(B, 2)
    tanimoto = jnp.array(1.0, dtype=jnp.float32)
    return (apd.reshape(B, nout), tanimoto, two_idx)


# speedup vs baseline: 1.4378x; 1.1545x over previous
import jax
import jax.numpy as jnp
from jax.experimental import pallas as pl

B, N, NF, EF = 64, 64, 128, 4
HID, MSG, T, ENN_H, GATH, MLP_H, FADD = 128, 64, 3, 64, 128, 128, 32

_MPNN_KEYS = ('W_embed', 'enn_W1', 'enn_b1', 'enn_W2', 'enn_b2', 'W_msg',
              'gru_Wi', 'gru_Wh', 'gru_bi', 'gru_bh',
              'att_W1', 'att_b1', 'att_W2', 'att_b2',
              'emb_W1', 'emb_b1', 'emb_W2', 'emb_b2')
_PG_KEYS = _MPNN_KEYS + ('mlp1_W1', 'mlp1_b1', 'mlp1_W2', 'mlp1_b2',
                         'mlp2_W1', 'mlp2_b1', 'mlp2_W2', 'mlp2_b2')
_PC_KEYS = _MPNN_KEYS + ('out_W1', 'out_b1', 'out_W2', 'out_b2')


def _dot(a, b):
    return jnp.dot(a, b, preferred_element_type=jnp.float32)


_TKEYS = frozenset({'W_msg', 'att_W1', 'emb_W1', 'mlp1_W2'})


def _dott(a, bt):
    return jax.lax.dot_general(a, bt, (((1,), (1,)), ((), ())),
                               preferred_element_type=jnp.float32)


def _tree_sum_axis1(x):
    while x.shape[1] > 1:
        half = x.shape[1] // 2
        x = x[:, :half] + x[:, half:]
    return x[:, 0]


def _gru_iters(h, gm3, p, streams):
    for _ in range(T):
        hj = _dott(h, p['W_msg'])
        if streams == 2:
            hj_pair = jnp.concatenate([hj[:N], hj[N:]], axis=1)
        else:
            hj_pair = hj
        m_pair = _tree_sum_axis1(gm3 * hj_pair[None, :, :])
        if streams == 2:
            m = jnp.concatenate([m_pair[:, :MSG], m_pair[:, MSG:]], axis=0)
        else:
            m = m_pair
        gi = _dot(m, p['gru_Wi']) + p['gru_bi']
        gh = _dot(h, p['gru_Wh']) + p['gru_bh']
        z = jax.nn.sigmoid(gi[:, :HID] + gh[:, :HID])
        r = jax.nn.sigmoid(gi[:, HID:2 * HID] + gh[:, HID:2 * HID])
        nmsg = jnp.tanh(gi[:, 2 * HID:] + r * gh[:, 2 * HID:])
        h = (1.0 - z) * nmsg + z * h
    return h


def _mpnn_pair(Xp, nodes_stack, p, W1p, b1p, W2p, b2p, Msum):
    h = _dot(nodes_stack, p['W_embed'])
    em = (_dot(jnp.abs(Xp), Msum) > 1e-6).astype(jnp.float32)
    a1 = jnp.maximum(_dot(Xp, W1p) + b1p, 0.0)
    gate = _dot(a1, W2p) + b2p
    gm3 = (gate * em).reshape(N, N, 2 * MSG)
    return _gru_iters(h, gm3, p, streams=2)


def _mpnn_single(X, nodes, p, Msum1):
    h = _dot(nodes, p['W_embed'])
    em = (_dot(jnp.abs(X), Msum1) > 1e-6).astype(jnp.float32)
    a1 = jnp.maximum(_dot(X, p['enn_W1']) + p['enn_b1'], 0.0)
    gate = _dot(a1, p['enn_W2']) + p['enn_b2']
    gm3 = (gate * em).reshape(N, N, MSG)
    return _gru_iters(h, gm3, p, streams=1)


def _tc_body(ln_ref, le_ref, fn_ref, xp_ref, *refs):
    npg, npc = len(_PG_KEYS), len(_PC_KEYS)
    pg = {k: refs[i][...] for i, k in enumerate(_PG_KEYS)}
    pc = {k: refs[npg + i][...] for i, k in enumerate(_PC_KEYS)}
    apd_ref, idx_ref = refs[npg + npc:]

    ln = ln_ref[0]
    fn = fn_ref[0]
    Xl = jnp.transpose(le_ref[0], (0, 2, 1)).reshape(N * N, EF)
    Xf = jnp.transpose(xp_ref[0], (0, 2, 1)).reshape(N * N, EF)
    Xp = jnp.concatenate([Xl, Xf], axis=1)

    z4 = jnp.zeros((EF, ENN_H), jnp.float32)
    zh = jnp.zeros((ENN_H, MSG), jnp.float32)
    w1p = jnp.concatenate([jnp.concatenate([pg['enn_W1'], z4], 1),
                           jnp.concatenate([z4, pg['enn_W1']], 1)], 0)
    w2p = jnp.concatenate([jnp.concatenate([pg['enn_W2'], zh], 1),
                           jnp.concatenate([zh, pg['enn_W2']], 1)], 0)
    b1p = jnp.concatenate([pg['enn_b1'], pg['enn_b1']], axis=1)
    b2p = jnp.concatenate([pg['enn_b2'], pg['enn_b2']], axis=1)
    o44 = jnp.ones((EF, MSG), jnp.float32)
    z44 = jnp.zeros((EF, MSG), jnp.float32)
    ms2 = jnp.concatenate([jnp.concatenate([o44, z44], 1),
                           jnp.concatenate([z44, o44], 1)], 0)

    nodes_stack = jnp.concatenate([ln, fn], axis=0)
    h_stack = _mpnn_pair(Xp, nodes_stack, pg, w1p, b1p, w2p, b2p, ms2)
    hl = h_stack[:N]

    cat = jnp.concatenate([h_stack, nodes_stack], axis=-1)
    att = jax.nn.sigmoid(
        _dot(jnp.maximum(_dott(cat, pg['att_W1']) + pg['att_b1'], 0.0),
             pg['att_W2']) + pg['att_b2'])
    emb = _dot(jnp.maximum(_dott(h_stack, pg['emb_W1']) + pg['emb_b1'], 0.0),
               pg['emb_W2']) + pg['emb_b2']
    ae = att * emb
    gl = jnp.sum(ae[:N], axis=0, keepdims=True)
    gf = jnp.sum(ae[N:], axis=0, keepdims=True)

    no = _dott(jnp.maximum(_dot(hl, pg['mlp1_W1']) + pg['mlp1_b1'], 0.0),
               pg['mlp1_W2']) + pg['mlp1_b2']
    na = no[:, :FADD]
    nc = no[:, FADD:FADD + EF]

    cat2 = jnp.concatenate([gl, gf], axis=-1)
    ft = _dot(jnp.maximum(_dot(cat2, pg['mlp2_W1']) + pg['mlp2_b1'], 0.0),
              pg['mlp2_W2']) + pg['mlp2_b2']

    mx = jnp.maximum(jnp.maximum(jnp.max(na), jnp.max(nc)), ft[0, 0])
    sa = jnp.exp(na - mx)
    sc = jnp.exp(nc - mx)
    st = jnp.exp(ft - mx)
    inv = 1.0 / (jnp.sum(sa) + jnp.sum(sc) + st[0, 0])
    ea = sa * inv
    ec = sc * inv
    for i in range(N):
        apd_ref[0, :, FADD * i:FADD * (i + 1)] = ea[i:i + 1, :]
        apd_ref[0, :, N * FADD + EF * i:N * FADD + EF * (i + 1)] = ec[i:i + 1, :]
    apd_ref[0, :, N * FADD + N * EF:] = st * inv

    hc = _mpnn_single(Xl, ln, pc, o44)
    co = _dot(jnp.maximum(_dot(hc, pc['out_W1']) + pc['out_b1'], 0.0),
              pc['out_W2']) + pc['out_b2']
    iot = jax.lax.broadcasted_iota(jnp.int32, (N, 1), 0)
    m1 = jnp.max(co, axis=0, keepdims=True)
    i1 = jnp.min(jnp.where(co >= m1, iot, N), axis=0, keepdims=True)
    co2 = jnp.where(iot == i1, -jnp.inf, co)
    m2 = jnp.max(co2, axis=0, keepdims=True)
    i2 = jnp.min(jnp.where(co2 >= m2, iot, N), axis=0, keepdims=True)
    idx_ref[0] = jnp.concatenate([i1, i2], axis=1)


def kernel(linker_nodes, linker_edges, fragment_nodes, fragment_edges,
           params_gen, params_con):
    def prep(params, k):
        x = params[k]
        if x.ndim == 1:
            return x.reshape(1, -1)
        return x.T if k in _TKEYS else x

    wg = [prep(params_gen, k) for k in _PG_KEYS]
    wc = [prep(params_con, k) for k in _PC_KEYS]
    let = jnp.transpose(linker_edges, (0, 1, 3, 2))
    fet = jnp.transpose(fragment_edges, (0, 1, 3, 2))

    def bspec(shape):
        nd = len(shape)
        return pl.BlockSpec((1,) + shape[1:], lambda b: (b,) + (0,) * (nd - 1))

    def wspec(x):
        nd = x.ndim
        return pl.BlockSpec(x.shape, lambda b: (0,) * nd)

    in_specs = [bspec((B, N, NF)), bspec((B, N, EF, N)),
                bspec((B, N, NF)), bspec((B, N, EF, N))]
    in_specs += [wspec(x) for x in wg + wc]

    nout = N * FADD + N * EF + 1
    out_shapes = [jax.ShapeDtypeStruct((B, 1, nout), jnp.float32),
                  jax.ShapeDtypeStruct((B, 1, 2), jnp.int32)]
    out_specs = [bspec((B, 1, nout)), bspec((B, 1, 2))]

    apd, idx = pl.pallas_call(
        _tc_body,
        grid=(B,),
        in_specs=in_specs,
        out_specs=out_specs,
        out_shape=out_shapes,
    )(linker_nodes, let, fragment_nodes, fet, *wg, *wc)

    two_idx = idx.reshape(B, 2)
    tanimoto = jnp.array(1.0, dtype=jnp.float32)
    return (apd.reshape(B, nout), tanimoto, two_idx)


# speedup vs baseline: 1.6377x; 1.1390x over previous
import jax
import jax.numpy as jnp
from jax.experimental import pallas as pl

B, N, NF, EF = 64, 64, 128, 4
HID, MSG, T, ENN_H, GATH, MLP_H, FADD = 128, 64, 3, 64, 128, 128, 32

_MPNN_KEYS = ('W_embed', 'enn_W1', 'enn_b1', 'enn_W2', 'enn_b2', 'W_msg',
              'gru_Wi', 'gru_Wh', 'gru_bi', 'gru_bh',
              'att_W1', 'att_b1', 'att_W2', 'att_b2',
              'emb_W1', 'emb_b1', 'emb_W2', 'emb_b2')
_PG_KEYS = _MPNN_KEYS + ('mlp1_W1', 'mlp1_b1', 'mlp1_W2', 'mlp1_b2',
                         'mlp2_W1', 'mlp2_b1', 'mlp2_W2', 'mlp2_b2')
_PC_KEYS = _MPNN_KEYS + ('out_W1', 'out_b1', 'out_W2', 'out_b2')


def _dot(a, b):
    return jnp.dot(a, b, preferred_element_type=jnp.float32)


_TKEYS = frozenset({'W_msg', 'att_W1', 'emb_W1', 'mlp1_W2'})


def _dott(a, bt):
    return jax.lax.dot_general(a, bt, (((1,), (1,)), ((), ())),
                               preferred_element_type=jnp.float32)


def _tree_sum_axis1(x):
    while x.shape[1] > 1:
        half = x.shape[1] // 2
        x = x[:, :half] + x[:, half:]
    return x[:, 0]


def _gru_iters(h, gm3, p, streams):
    for _ in range(T):
        hj = _dott(h, p['W_msg'])
        if streams == 2:
            hj_pair = jnp.concatenate([hj[:N], hj[N:]], axis=1)
        else:
            hj_pair = hj
        m_pair = _tree_sum_axis1(gm3 * hj_pair[None, :, :])
        if streams == 2:
            m = jnp.concatenate([m_pair[:, :MSG], m_pair[:, MSG:]], axis=0)
        else:
            m = m_pair
        gi = _dot(m, p['gru_Wi']) + p['gru_bi']
        gh = _dot(h, p['gru_Wh']) + p['gru_bh']
        z = jax.nn.sigmoid(gi[:, :HID] + gh[:, :HID])
        r = jax.nn.sigmoid(gi[:, HID:2 * HID] + gh[:, HID:2 * HID])
        nmsg = jnp.tanh(gi[:, 2 * HID:] + r * gh[:, 2 * HID:])
        h = (1.0 - z) * nmsg + z * h
    return h


def _mpnn_pair(Xp, nodes_stack, p, W1p, b1p, W2p, b2p, Msum):
    h = _dot(nodes_stack, p['W_embed'])
    es = _dot(jnp.abs(Xp), Msum)
    a1 = jnp.maximum(_dot(Xp, W1p) + b1p, 0.0)
    gate = _dot(a1, W2p) + b2p
    gm3 = jnp.where(es > 1e-6, gate, 0.0).reshape(N, N, 2 * MSG)
    return _gru_iters(h, gm3, p, streams=2)


def _mpnn_single(X, nodes, p, Msum1):
    h = _dot(nodes, p['W_embed'])
    es = _dot(jnp.abs(X), Msum1)
    a1 = jnp.maximum(_dot(X, p['enn_W1']) + p['enn_b1'], 0.0)
    gate = _dot(a1, p['enn_W2']) + p['enn_b2']
    gm3 = jnp.where(es > 1e-6, gate, 0.0).reshape(N, N, MSG)
    return _gru_iters(h, gm3, p, streams=1)


def _tc_body(ln_ref, le_ref, fn_ref, xp_ref, *refs):
    npg, npc = len(_PG_KEYS), len(_PC_KEYS)
    pg = {k: refs[i][...] for i, k in enumerate(_PG_KEYS)}
    pc = {k: refs[npg + i][...] for i, k in enumerate(_PC_KEYS)}
    apd_ref, idx_ref = refs[npg + npc:]

    ln = ln_ref[0]
    fn = fn_ref[0]
    xcat = jnp.concatenate([le_ref[0], xp_ref[0]], axis=1)
    Xp = jnp.transpose(xcat, (0, 2, 1)).reshape(N * N, 2 * EF)
    Xl = Xp[:, :EF]

    z4 = jnp.zeros((EF, ENN_H), jnp.float32)
    zh = jnp.zeros((ENN_H, MSG), jnp.float32)
    w1p = jnp.concatenate([jnp.concatenate([pg['enn_W1'], z4], 1),
                           jnp.concatenate([z4, pg['enn_W1']], 1)], 0)
    w2p = jnp.concatenate([jnp.concatenate([pg['enn_W2'], zh], 1),
                           jnp.concatenate([zh, pg['enn_W2']], 1)], 0)
    b1p = jnp.concatenate([pg['enn_b1'], pg['enn_b1']], axis=1)
    b2p = jnp.concatenate([pg['enn_b2'], pg['enn_b2']], axis=1)
    o44 = jnp.ones((EF, MSG), jnp.float32)
    z44 = jnp.zeros((EF, MSG), jnp.float32)
    ms2 = jnp.concatenate([jnp.concatenate([o44, z44], 1),
                           jnp.concatenate([z44, o44], 1)], 0)

    nodes_stack = jnp.concatenate([ln, fn], axis=0)
    h_stack = _mpnn_pair(Xp, nodes_stack, pg, w1p, b1p, w2p, b2p, ms2)
    hl = h_stack[:N]

    cat = jnp.concatenate([h_stack, nodes_stack], axis=-1)
    att = jax.nn.sigmoid(
        _dot(jnp.maximum(_dott(cat, pg['att_W1']) + pg['att_b1'], 0.0),
             pg['att_W2']) + pg['att_b2'])
    emb = _dot(jnp.maximum(_dott(h_stack, pg['emb_W1']) + pg['emb_b1'], 0.0),
               pg['emb_W2']) + pg['emb_b2']
    ae = att * emb
    gl = jnp.sum(ae[:N], axis=0, keepdims=True)
    gf = jnp.sum(ae[N:], axis=0, keepdims=True)

    no = _dott(jnp.maximum(_dot(hl, pg['mlp1_W1']) + pg['mlp1_b1'], 0.0),
               pg['mlp1_W2']) + pg['mlp1_b2']
    na = no[:, :FADD]
    nc = no[:, FADD:FADD + EF]

    cat2 = jnp.concatenate([gl, gf], axis=-1)
    ft = _dot(jnp.maximum(_dot(cat2, pg['mlp2_W1']) + pg['mlp2_b1'], 0.0),
              pg['mlp2_W2']) + pg['mlp2_b2']

    mx = jnp.maximum(jnp.maximum(jnp.max(na), jnp.max(nc)), ft[0, 0])
    sa = jnp.exp(na - mx)
    sc = jnp.exp(nc - mx)
    st = jnp.exp(ft - mx)
    inv = 1.0 / (jnp.sum(sa) + jnp.sum(sc) + st[0, 0])
    ea = sa * inv
    ec = sc * inv
    for i in range(N):
        apd_ref[0, :, FADD * i:FADD * (i + 1)] = ea[i:i + 1, :]
        apd_ref[0, :, N * FADD + EF * i:N * FADD + EF * (i + 1)] = ec[i:i + 1, :]
    apd_ref[0, :, N * FADD + N * EF:] = st * inv

    hc = _mpnn_single(Xl, ln, pc, o44)
    co = _dot(jnp.maximum(_dot(hc, pc['out_W1']) + pc['out_b1'], 0.0),
              pc['out_W2']) + pc['out_b2']
    iot = jax.lax.broadcasted_iota(jnp.int32, (N, 1), 0)
    m1 = jnp.max(co, axis=0, keepdims=True)
    i1 = jnp.min(jnp.where(co >= m1, iot, N), axis=0, keepdims=True)
    co2 = jnp.where(iot == i1, -jnp.inf, co)
    m2 = jnp.max(co2, axis=0, keepdims=True)
    i2 = jnp.min(jnp.where(co2 >= m2, iot, N), axis=0, keepdims=True)
    idx_ref[0] = jnp.concatenate([i1, i2], axis=1)


def kernel(linker_nodes, linker_edges, fragment_nodes, fragment_edges,
           params_gen, params_con):
    def prep(params, k):
        x = params[k]
        if x.ndim == 1:
            return x.reshape(1, -1)
        return x.T if k in _TKEYS else x

    wg = [prep(params_gen, k) for k in _PG_KEYS]
    wc = [prep(params_con, k) for k in _PC_KEYS]
    let = jnp.transpose(linker_edges, (0, 1, 3, 2))
    fet = jnp.transpose(fragment_edges, (0, 1, 3, 2))

    def bspec(shape):
        nd = len(shape)
        return pl.BlockSpec((1,) + shape[1:], lambda b: (b,) + (0,) * (nd - 1))

    def wspec(x):
        nd = x.ndim
        return pl.BlockSpec(x.shape, lambda b: (0,) * nd)

    in_specs = [bspec((B, N, NF)), bspec((B, N, EF, N)),
                bspec((B, N, NF)), bspec((B, N, EF, N))]
    in_specs += [wspec(x) for x in wg + wc]

    nout = N * FADD + N * EF + 1
    out_shapes = [jax.ShapeDtypeStruct((B, 1, nout), jnp.float32),
                  jax.ShapeDtypeStruct((B, 1, 2), jnp.int32)]
    out_specs = [bspec((B, 1, nout)), bspec((B, 1, 2))]

    apd, idx = pl.pallas_call(
        _tc_body,
        grid=(B,),
        in_specs=in_specs,
        out_specs=out_specs,
        out_shape=out_shapes,
    )(linker_nodes, let, fragment_nodes, fet, *wg, *wc)

    two_idx = idx.reshape(B, 2)
    tanimoto = jnp.array(1.0, dtype=jnp.float32)
    return (apd.reshape(B, nout), tanimoto, two_idx)
